# baseline (speedup 1.0000x reference)
import functools

import jax
import jax.numpy as jnp
from jax import lax
from jax.experimental import pallas as pl
from jax.experimental.pallas import tpu as pltpu
from jax.experimental.pallas import tpu_sc as plsc

VOCAB = 100000
D = 64
BATCH = 4096
SEQ = 200

NC = 2
NS = 16
NW = NC * NS

BBLK = BATCH // NW
LANES = 16
NBB = BBLK // LANES


def _pos_encoding():
    even_i = jnp.arange(0, D, 2).astype(jnp.float32)
    denominator = jnp.power(10000.0, even_i / D)
    position = jnp.arange(SEQ).reshape(SEQ, 1).astype(jnp.float32)
    even_pe = jnp.sin(position / denominator)
    odd_pe = jnp.cos(position / denominator)
    return jnp.stack([even_pe, odd_pe], axis=2).reshape(SEQ, D)


_mesh = plsc.VectorSubcoreMesh(core_axis_name="c", subcore_axis_name="s")


@functools.partial(
    pl.kernel,
    mesh=_mesh,
    compiler_params=pltpu.CompilerParams(use_tc_tiling_on_sc=False,
                                         needs_layout_passes=False),
    out_type=jax.ShapeDtypeStruct((SEQ, D // 8, BATCH // BBLK, 8, BBLK),
                                  jnp.float32),
    scratch_types=[
        pltpu.VMEM((SEQ, BBLK), jnp.int32),
        pltpu.VMEM((SEQ, D), jnp.float32),
        pltpu.VMEM((BBLK, D), jnp.float32),
        pltpu.VMEM((BBLK, D), jnp.float32),
        pltpu.VMEM((D // 8, 8, BBLK), jnp.float32),
        pltpu.VMEM((D // 8, 8, BBLK), jnp.float32),
        pltpu.SemaphoreType.DMA,
        pltpu.SemaphoreType.DMA,
        pltpu.SemaphoreType.DMA,
        pltpu.SemaphoreType.DMA,
    ],
)
def _emb_kernel(idx_hbm, table_hbm, pe_hbm, out_hbm,
                idx_v, pe_v, gbuf0, gbuf1, tbuf0, tbuf1,
                semg0, semg1, semo0, semo1):
    wid = lax.axis_index("s") * NC + lax.axis_index("c")

    pltpu.sync_copy(idx_hbm.at[:, pl.ds(wid * BBLK, BBLK)], idx_v)
    pltpu.sync_copy(pe_hbm, pe_v)

    gbufs = (gbuf0, gbuf1)
    tbufs = (tbuf0, tbuf1)
    semgs = (semg0, semg1)
    semos = (semo0, semo1)

    iota = lax.broadcasted_iota(jnp.int32, (LANES,), 0)

    pltpu.async_copy(table_hbm.at[idx_v.at[0]], gbuf0, semg0)
    pltpu.async_copy(table_hbm.at[idx_v.at[1]], gbuf1, semg1)

    def outer(i, carry):
        l0 = i * 2
        for b in range(2):
            l = l0 + b
            gbuf, tbuf = gbufs[b], tbufs[b]
            semg, semo = semgs[b], semos[b]

            pltpu.make_async_copy(table_hbm.at[idx_v.at[l]], gbuf, semg).wait()
            @pl.when(l >= 2)
            def _drain(_tbuf=tbuf, _semo=semo):
                pltpu.make_async_copy(_tbuf, out_hbm.at[0, :, wid],
                                      _semo).wait()

            def col(c, cc, _gbuf=gbuf, _tbuf=tbuf, _l=l):
                cols = jnp.full((LANES,), c, jnp.int32)
                pe_c = plsc.load_gather(pe_v, [jnp.full((LANES,), _l, jnp.int32), cols])
                for bb in range(NBB):
                    rows = bb * LANES + iota
                    v = plsc.load_gather(_gbuf, [rows, cols]) + pe_c
                    _tbuf[c // 8, c % 8, pl.ds(bb * LANES, LANES)] = v
                return cc

            lax.fori_loop(0, D, col, 0)

            pltpu.async_copy(tbuf, out_hbm.at[l, :, wid], semo)

            @pl.when(l + 2 < SEQ)
            def _refill(_gbuf=gbuf, _semg=semg, _l=l):
                pltpu.async_copy(table_hbm.at[idx_v.at[_l + 2]], _gbuf, _semg)

        return carry

    lax.fori_loop(0, SEQ // 2, outer, 0)

    pltpu.make_async_copy(tbuf0, out_hbm.at[0, :, wid], semo0).wait()
    pltpu.make_async_copy(tbuf1, out_hbm.at[0, :, wid], semo1).wait()


def kernel(indices, table):
    pe = _pos_encoding()
    idx_t = indices.astype(jnp.int32).T
    out5 = _emb_kernel(idx_t, table, pe)
    return out5.transpose((2, 4, 0, 1, 3)).reshape(BATCH, SEQ, D)


# speedup vs baseline: 2.4493x; 2.4493x over previous
import functools

import jax
import jax.numpy as jnp
from jax import lax
from jax.experimental import pallas as pl
from jax.experimental.pallas import tpu as pltpu
from jax.experimental.pallas import tpu_sc as plsc

VOCAB = 100000
D = 64
BATCH = 4096
SEQ = 200

NC = 2
NS = 16
NW = NC * NS

BBLK = BATCH // NW
LANES = 16
NBB = BBLK // LANES
TPITCH = 133


def _pos_encoding():
    even_i = jnp.arange(0, D, 2).astype(jnp.float32)
    denominator = jnp.power(10000.0, even_i / D)
    position = jnp.arange(SEQ).reshape(SEQ, 1).astype(jnp.float32)
    even_pe = jnp.sin(position / denominator)
    odd_pe = jnp.cos(position / denominator)
    return jnp.stack([even_pe, odd_pe], axis=2).reshape(SEQ, D)


_mesh = plsc.VectorSubcoreMesh(core_axis_name="c", subcore_axis_name="s")


@functools.partial(
    pl.kernel,
    mesh=_mesh,
    compiler_params=pltpu.CompilerParams(use_tc_tiling_on_sc=False,
                                         needs_layout_passes=False),
    out_type=jax.ShapeDtypeStruct((SEQ, D // 8, BATCH // BBLK, 8, BBLK),
                                  jnp.float32),
    scratch_types=[
        pltpu.VMEM((SEQ, BBLK), jnp.int32),
        pltpu.VMEM((SEQ, D), jnp.float32),
        pltpu.VMEM((BBLK, D), jnp.float32),
        pltpu.VMEM((BBLK, D), jnp.float32),
        pltpu.VMEM((D // 8, 8, TPITCH), jnp.float32),
        pltpu.VMEM((D // 8, 8, TPITCH), jnp.float32),
        pltpu.SemaphoreType.DMA,
        pltpu.SemaphoreType.DMA,
        pltpu.SemaphoreType.DMA,
        pltpu.SemaphoreType.DMA,
    ],
)
def _emb_kernel(idx_hbm, table_hbm, pe_hbm, out_hbm,
                idx_v, pe_v, gbuf0, gbuf1, tbuf0, tbuf1,
                semg0, semg1, semo0, semo1):
    wid = lax.axis_index("s") * NC + lax.axis_index("c")

    pltpu.sync_copy(idx_hbm.at[:, pl.ds(wid * BBLK, BBLK)], idx_v)
    pltpu.sync_copy(pe_hbm, pe_v)

    gbufs = (gbuf0, gbuf1)
    tbufs = (tbuf0, tbuf1)
    semgs = (semg0, semg1)
    semos = (semo0, semo1)

    iota = lax.broadcasted_iota(jnp.int32, (LANES,), 0)
    scat_d0 = [(k * LANES + iota) // 8 for k in range(D // LANES)]
    scat_d1 = [(k * LANES + iota) % 8 for k in range(D // LANES)]

    pltpu.async_copy(table_hbm.at[idx_v.at[0]], gbuf0, semg0)
    pltpu.async_copy(table_hbm.at[idx_v.at[1]], gbuf1, semg1)

    def outer(i, carry):
        l0 = i * 2
        for b in range(2):
            l = l0 + b
            gbuf, tbuf = gbufs[b], tbufs[b]
            semg, semo = semgs[b], semos[b]

            pltpu.make_async_copy(table_hbm.at[idx_v.at[l]], gbuf, semg).wait()
            @pl.when(l >= 2)
            def _drain(_tbuf=tbuf, _semo=semo):
                pltpu.make_async_copy(_tbuf.at[:, :, pl.ds(0, BBLK)],
                                      out_hbm.at[0, :, wid], _semo).wait()

            pe_rows = [pe_v[l, pl.ds(k * LANES, LANES)] for k in range(D // LANES)]

            def row(r, cc, _gbuf=gbuf, _tbuf=tbuf):
                rvec = jnp.full((LANES,), r, jnp.int32)
                for k in range(D // LANES):
                    v = _gbuf[r, pl.ds(k * LANES, LANES)] + pe_rows[k]
                    plsc.store_scatter(_tbuf, [scat_d0[k], scat_d1[k], rvec], v)
                return cc

            lax.fori_loop(0, BBLK, row, 0, unroll=4)

            pltpu.async_copy(tbuf.at[:, :, pl.ds(0, BBLK)],
                             out_hbm.at[l, :, wid], semo)

            @pl.when(l + 2 < SEQ)
            def _refill(_gbuf=gbuf, _semg=semg, _l=l):
                pltpu.async_copy(table_hbm.at[idx_v.at[_l + 2]], _gbuf, _semg)

        return carry

    lax.fori_loop(0, SEQ // 2, outer, 0)

    pltpu.make_async_copy(tbuf0.at[:, :, pl.ds(0, BBLK)],
                          out_hbm.at[0, :, wid], semo0).wait()
    pltpu.make_async_copy(tbuf1.at[:, :, pl.ds(0, BBLK)],
                          out_hbm.at[0, :, wid], semo1).wait()


def kernel(indices, table):
    pe = _pos_encoding()
    idx_t = indices.astype(jnp.int32).T
    out5 = _emb_kernel(idx_t, table, pe)
    return out5.transpose((2, 4, 0, 1, 3)).reshape(BATCH, SEQ, D)
